# Initial kernel scaffold; baseline (speedup 1.0000x reference)
import jax
import jax.numpy as jnp
from jax.experimental import pallas as pl


def kernel(x, params):
    raise NotImplementedError("write your pallas kernel here")


# speedup vs baseline: 11.6243x; 11.6243x over previous
import functools

import numpy as np
import jax
import jax.numpy as jnp
from jax.experimental import pallas as pl

_B, _N, _D = 4, 2048, 512
_H, _HD, _K = 8, 64, 16
_EPS = 1e-5
_BN = float(1.0 / np.sqrt(1.0 + 1e-5))
_TP = 512
_TA = 256
_F32 = jnp.float32


def _ln(h, g, b):
    mu = jnp.mean(h, axis=-1, keepdims=True)
    c = h - mu
    var = jnp.mean(c * c, axis=-1, keepdims=True)
    return c / jnp.sqrt(var + _EPS) * g + b


def _dot(a, b):
    return jnp.dot(a, b, preferred_element_type=_F32)


def _dot_t(a, b):
    return jax.lax.dot_general(a, b, (((1,), (1,)), ((), ())),
                               preferred_element_type=_F32)


def _embed_body(x_ref, w1_ref, b1_ref, g1_ref, e1_ref, w2_ref, b2_ref,
                g2_ref, e2_ref, o_ref):
    h = _dot(x_ref[...], w1_ref[...]) + b1_ref[...]
    h = jnp.maximum(h * _BN * g1_ref[...] + e1_ref[...], 0.0)
    h = _dot(h, w2_ref[...]) + b2_ref[...]
    h = jnp.maximum(h * _BN * g2_ref[...] + e2_ref[...], 0.0)
    o_ref[...] = h


def _proj_body(hin_ref, win_ref, bin_ref, gin_ref, ein_ref,
               wq_ref, bq_ref, wk_ref, bk_ref, wv_ref, bv_ref,
               h_ref, q_ref, k_ref, v_ref):
    hw = _dot(hin_ref[...], win_ref[...]) + bin_ref[...]
    h = _ln(hw, gin_ref[...], ein_ref[...])
    h_ref[...] = h
    q_ref[...] = _dot(h, wq_ref[...]) + bq_ref[...]
    k_ref[...] = _dot(h, wk_ref[...]) + bk_ref[...]
    v_ref[...] = _dot(h, wv_ref[...]) + bv_ref[...]


def _attn_body(hf_ref, ht_ref, q_ref, kf_ref, vf_ref, ga_ref, ba_ref, o_ref):
    hq = ht_ref[0]
    hall = hf_ref[0]

    sq = hall * hall
    sqrow = _dot_t(jnp.ones((8, _D), _F32), sq)[0:1]
    dist = sqrow - 2.0 * _dot_t(hq, hall)

    iota = jax.lax.broadcasted_iota(jnp.int32, (_TA, _N), 1)
    work = dist
    msk = jnp.zeros((_TA, _N), jnp.bool_)
    for _ in range(_K):
        m = jnp.min(work, axis=1, keepdims=True)
        cand = jnp.where(work == m, iota, _N)
        jstar = jnp.min(cand, axis=1, keepdims=True)
        sel = iota == jstar
        msk = jnp.logical_or(msk, sel)
        work = jnp.where(sel, jnp.float32(np.inf), work)

    q = q_ref[0]
    neginf = jnp.float32(-np.inf)
    outs = []
    for hh in range(_H):
        sl = slice(hh * _HD, (hh + 1) * _HD)
        lg = _dot_t(q[:, sl], kf_ref[0, :, sl]) * (1.0 / 8.0)
        lg = jnp.where(msk, lg, neginf)
        mx = jnp.max(lg, axis=1, keepdims=True)
        e = jnp.exp(lg - mx)
        p = e / jnp.sum(e, axis=1, keepdims=True)
        outs.append(_dot(p, vf_ref[0, :, sl]))
    out = jnp.concatenate(outs, axis=1)
    o_ref[0] = _ln(out + hq, ga_ref[...], ba_ref[...])


def _ffn_body(h_ref, w1_ref, b1_ref, w2_ref, b2_ref, g_ref, e_ref, o_ref):
    h = h_ref[...]
    f = jnp.maximum(_dot(h, w1_ref[...]) + b1_ref[...], 0.0)
    f = _dot(f, w2_ref[...]) + b2_ref[...]
    o_ref[...] = _ln(f + h, g_ref[...], e_ref[...])


def _head_body(h_ref, w1_ref, b1_ref, g1_ref, e1_ref, w2_ref, b2_ref,
               g2_ref, e2_ref, w3_ref, b3_ref, o_ref):
    gs = [jnp.max(h_ref[b], axis=0, keepdims=True) for b in range(_B)]
    g = jnp.concatenate(gs + gs, axis=0)
    z = _dot(g, w1_ref[...]) + b1_ref[...]
    z = jnp.maximum(z * _BN * g1_ref[...] + e1_ref[...], 0.0)
    z = _dot(z, w2_ref[...]) + b2_ref[...]
    z = jnp.maximum(z * _BN * g2_ref[...] + e2_ref[...], 0.0)
    o_ref[...] = _dot(z, w3_ref[...]) + b3_ref[...]


def _full(shape):
    return pl.BlockSpec(shape, lambda *_: tuple(0 for _ in shape))


def _row(c):
    return pl.BlockSpec((_TP, c), lambda i: (i, 0))


def kernel(x, params):
    emb, blocks, fc = params['emb'], params['blocks'], params['fc']
    rows = _B * _N

    xp = jnp.pad(x.reshape(rows, 3), ((0, 0), (0, 125)))
    w1 = jnp.pad(emb['W1'].T, ((0, 125), (0, 0)))
    h = pl.pallas_call(
        _embed_body,
        grid=(rows // _TP,),
        in_specs=[_row(128), _full((128, 128)), _full((1, 128)),
                  _full((1, 128)), _full((1, 128)), _full((128, 256)),
                  _full((1, 256)), _full((1, 256)), _full((1, 256))],
        out_specs=_row(256),
        out_shape=jax.ShapeDtypeStruct((rows, 256), _F32),
    )(xp, w1, emb['b1'][None], emb['g1'][None], emb['be1'][None],
      emb['W2'].T, emb['b2'][None], emb['g2'][None], emb['be2'][None])

    for blk in blocks:
        c_in = h.shape[-1]
        h_, q, k, v = pl.pallas_call(
            _proj_body,
            grid=(rows // _TP,),
            in_specs=[_row(c_in), _full((c_in, _D)), _full((1, _D)),
                      _full((1, _D)), _full((1, _D)),
                      _full((_D, _D)), _full((1, _D)),
                      _full((_D, _D)), _full((1, _D)),
                      _full((_D, _D)), _full((1, _D))],
            out_specs=[_row(_D)] * 4,
            out_shape=[jax.ShapeDtypeStruct((rows, _D), _F32)] * 4,
        )(h, blk['Win'].T, blk['bin'][None], blk['g_in'][None],
          blk['b_in'][None], blk['Wq'].T, blk['bq'][None],
          blk['Wk'].T, blk['bk'][None], blk['Wv'].T, blk['bv'][None])

        h3 = h_.reshape(_B, _N, _D)
        q3 = q.reshape(_B, _N, _D)
        k3 = k.reshape(_B, _N, _D)
        v3 = v.reshape(_B, _N, _D)

        bfull = pl.BlockSpec((1, _N, _D), lambda b, t: (b, 0, 0))
        btile = pl.BlockSpec((1, _TA, _D), lambda b, t: (b, t, 0))
        bvec = pl.BlockSpec((1, _D), lambda b, t: (0, 0))
        h2 = pl.pallas_call(
            _attn_body,
            grid=(_B, _N // _TA),
            in_specs=[bfull, btile, btile, bfull, bfull, bvec, bvec],
            out_specs=btile,
            out_shape=jax.ShapeDtypeStruct((_B, _N, _D), _F32),
        )(h3, h3, q3, k3, v3, blk['g_attn'][None], blk['b_attn'][None])

        h = pl.pallas_call(
            _ffn_body,
            grid=(rows // _TP,),
            in_specs=[_row(_D), _full((_D, 4 * _D)), _full((1, 4 * _D)),
                      _full((4 * _D, _D)), _full((1, _D)),
                      _full((1, _D)), _full((1, _D))],
            out_specs=_row(_D),
            out_shape=jax.ShapeDtypeStruct((rows, _D), _F32),
        )(h2.reshape(rows, _D), blk['Wf1'].T, blk['bf1'][None],
          blk['Wf2'].T, blk['bf2'][None], blk['g_ffn'][None],
          blk['b_ffn'][None])

    w3p = jnp.pad(fc['W3'].T, ((0, 0), (0, 128 - 40)))
    b3p = jnp.pad(fc['b3'][None], ((0, 0), (0, 128 - 40)))
    logits = pl.pallas_call(
        _head_body,
        grid=(1,),
        in_specs=[pl.BlockSpec((_B, _N, _D), lambda i: (0, 0, 0)),
                  _full((_D, _D)), _full((1, _D)), _full((1, _D)),
                  _full((1, _D)), _full((_D, 256)), _full((1, 256)),
                  _full((1, 256)), _full((1, 256)), _full((256, 128)),
                  _full((1, 128))],
        out_specs=_full((8, 128)),
        out_shape=jax.ShapeDtypeStruct((8, 128), _F32),
    )(h.reshape(_B, _N, _D), fc['W1'].T, fc['b1'][None], fc['g1'][None],
      fc['be1'][None], fc['W2'].T, fc['b2'][None], fc['g2'][None],
      fc['be2'][None], w3p, b3p)

    return logits[:_B, :40]


# speedup vs baseline: 16.7312x; 1.4393x over previous
import functools

import numpy as np
import jax
import jax.numpy as jnp
from jax.experimental import pallas as pl

_B, _N, _D = 4, 2048, 512
_H, _HD, _K = 8, 64, 16
_EPS = 1e-5
_BN = float(1.0 / np.sqrt(1.0 + 1e-5))
_TP = 512
_TA = 256
_F32 = jnp.float32


def _ln(h, g, b):
    mu = jnp.mean(h, axis=-1, keepdims=True)
    c = h - mu
    var = jnp.mean(c * c, axis=-1, keepdims=True)
    return c / jnp.sqrt(var + _EPS) * g + b


def _dot(a, b):
    return jnp.dot(a, b, preferred_element_type=_F32)


def _dot_t(a, b):
    return jax.lax.dot_general(a, b, (((1,), (1,)), ((), ())),
                               preferred_element_type=_F32)


def _embed_body(x_ref, w1_ref, b1_ref, g1_ref, e1_ref, w2_ref, b2_ref,
                g2_ref, e2_ref, o_ref):
    h = _dot(x_ref[...], w1_ref[...]) + b1_ref[...]
    h = jnp.maximum(h * _BN * g1_ref[...] + e1_ref[...], 0.0)
    h = _dot(h, w2_ref[...]) + b2_ref[...]
    h = jnp.maximum(h * _BN * g2_ref[...] + e2_ref[...], 0.0)
    o_ref[...] = h


def _proj_body(hin_ref, win_ref, bin_ref, gin_ref, ein_ref,
               wq_ref, bq_ref, wk_ref, bk_ref, wv_ref, bv_ref,
               h_ref, q_ref, k_ref, v_ref):
    hw = _dot(hin_ref[...].astype(jnp.bfloat16), win_ref[...]) + bin_ref[...]
    h = _ln(hw, gin_ref[...], ein_ref[...])
    h_ref[...] = h
    hb = h.astype(jnp.bfloat16)
    q_ref[...] = _dot(hb, wq_ref[...]) + bq_ref[...]
    k_ref[...] = _dot(hb, wk_ref[...]) + bk_ref[...]
    v_ref[...] = _dot(hb, wv_ref[...]) + bv_ref[...]


def _attn_body(hf_ref, ht_ref, q_ref, kf_ref, vf_ref, ga_ref, ba_ref, o_ref):
    hq = ht_ref[0]
    hall = hf_ref[0]

    sq = hall * hall
    sqrow = _dot_t(jnp.ones((8, _D), _F32), sq)[0:1]
    dist = sqrow - 2.0 * _dot_t(hq, hall)

    work = dist
    m = jnp.zeros((_TA, 1), _F32)
    for _ in range(_K):
        m = jnp.min(work, axis=1, keepdims=True)
        work = jnp.where(work == m, jnp.float32(np.inf), work)
    msk = dist <= m

    q = q_ref[0].astype(jnp.bfloat16)
    kb = kf_ref[0].astype(jnp.bfloat16)
    vb = vf_ref[0].astype(jnp.bfloat16)
    neginf = jnp.float32(-np.inf)
    outs = []
    for hh in range(_H):
        sl = slice(hh * _HD, (hh + 1) * _HD)
        lg = _dot_t(q[:, sl], kb[:, sl]) * (1.0 / 8.0)
        lg = jnp.where(msk, lg, neginf)
        mx = jnp.max(lg, axis=1, keepdims=True)
        e = jnp.exp(lg - mx)
        p = (e / jnp.sum(e, axis=1, keepdims=True)).astype(jnp.bfloat16)
        outs.append(_dot(p, vb[:, sl]))
    out = jnp.concatenate(outs, axis=1)
    o_ref[0] = _ln(out + hq, ga_ref[...], ba_ref[...])


def _ffn_body(h_ref, w1_ref, b1_ref, w2_ref, b2_ref, g_ref, e_ref, o_ref):
    h = h_ref[...]
    f = jnp.maximum(_dot(h.astype(jnp.bfloat16), w1_ref[...]) + b1_ref[...],
                    0.0)
    f = _dot(f.astype(jnp.bfloat16), w2_ref[...]) + b2_ref[...]
    o_ref[...] = _ln(f + h, g_ref[...], e_ref[...])


def _head_body(h_ref, w1_ref, b1_ref, g1_ref, e1_ref, w2_ref, b2_ref,
               g2_ref, e2_ref, w3_ref, b3_ref, o_ref):
    gs = [jnp.max(h_ref[b], axis=0, keepdims=True) for b in range(_B)]
    g = jnp.concatenate(gs + gs, axis=0)
    z = _dot(g, w1_ref[...]) + b1_ref[...]
    z = jnp.maximum(z * _BN * g1_ref[...] + e1_ref[...], 0.0)
    z = _dot(z, w2_ref[...]) + b2_ref[...]
    z = jnp.maximum(z * _BN * g2_ref[...] + e2_ref[...], 0.0)
    o_ref[...] = _dot(z, w3_ref[...]) + b3_ref[...]


def _full(shape):
    return pl.BlockSpec(shape, lambda *_: tuple(0 for _ in shape))


def _row(c):
    return pl.BlockSpec((_TP, c), lambda i: (i, 0))


def kernel(x, params):
    emb, blocks, fc = params['emb'], params['blocks'], params['fc']
    rows = _B * _N

    xp = jnp.pad(x.reshape(rows, 3), ((0, 0), (0, 125)))
    w1 = jnp.pad(emb['W1'].T, ((0, 125), (0, 0)))
    h = pl.pallas_call(
        _embed_body,
        grid=(rows // _TP,),
        in_specs=[_row(128), _full((128, 128)), _full((1, 128)),
                  _full((1, 128)), _full((1, 128)), _full((128, 256)),
                  _full((1, 256)), _full((1, 256)), _full((1, 256))],
        out_specs=_row(256),
        out_shape=jax.ShapeDtypeStruct((rows, 256), _F32),
    )(xp, w1, emb['b1'][None], emb['g1'][None], emb['be1'][None],
      emb['W2'].T, emb['b2'][None], emb['g2'][None], emb['be2'][None])

    for blk in blocks:
        c_in = h.shape[-1]
        h_, q, k, v = pl.pallas_call(
            _proj_body,
            grid=(rows // _TP,),
            in_specs=[_row(c_in), _full((c_in, _D)), _full((1, _D)),
                      _full((1, _D)), _full((1, _D)),
                      _full((_D, _D)), _full((1, _D)),
                      _full((_D, _D)), _full((1, _D)),
                      _full((_D, _D)), _full((1, _D))],
            out_specs=[_row(_D)] * 4,
            out_shape=[jax.ShapeDtypeStruct((rows, _D), _F32)] * 4,
        )(h, blk['Win'].T.astype(jnp.bfloat16), blk['bin'][None],
          blk['g_in'][None], blk['b_in'][None],
          blk['Wq'].T.astype(jnp.bfloat16), blk['bq'][None],
          blk['Wk'].T.astype(jnp.bfloat16), blk['bk'][None],
          blk['Wv'].T.astype(jnp.bfloat16), blk['bv'][None])

        h3 = h_.reshape(_B, _N, _D)
        q3 = q.reshape(_B, _N, _D)
        k3 = k.reshape(_B, _N, _D)
        v3 = v.reshape(_B, _N, _D)

        bfull = pl.BlockSpec((1, _N, _D), lambda b, t: (b, 0, 0))
        btile = pl.BlockSpec((1, _TA, _D), lambda b, t: (b, t, 0))
        bvec = pl.BlockSpec((1, _D), lambda b, t: (0, 0))
        h2 = pl.pallas_call(
            _attn_body,
            grid=(_B, _N // _TA),
            in_specs=[bfull, btile, btile, bfull, bfull, bvec, bvec],
            out_specs=btile,
            out_shape=jax.ShapeDtypeStruct((_B, _N, _D), _F32),
        )(h3, h3, q3, k3, v3, blk['g_attn'][None], blk['b_attn'][None])

        h = pl.pallas_call(
            _ffn_body,
            grid=(rows // _TP,),
            in_specs=[_row(_D), _full((_D, 4 * _D)), _full((1, 4 * _D)),
                      _full((4 * _D, _D)), _full((1, _D)),
                      _full((1, _D)), _full((1, _D))],
            out_specs=_row(_D),
            out_shape=jax.ShapeDtypeStruct((rows, _D), _F32),
        )(h2.reshape(rows, _D), blk['Wf1'].T.astype(jnp.bfloat16),
          blk['bf1'][None], blk['Wf2'].T.astype(jnp.bfloat16),
          blk['bf2'][None], blk['g_ffn'][None], blk['b_ffn'][None])

    w3p = jnp.pad(fc['W3'].T, ((0, 0), (0, 128 - 40)))
    b3p = jnp.pad(fc['b3'][None], ((0, 0), (0, 128 - 40)))
    logits = pl.pallas_call(
        _head_body,
        grid=(1,),
        in_specs=[pl.BlockSpec((_B, _N, _D), lambda i: (0, 0, 0)),
                  _full((_D, _D)), _full((1, _D)), _full((1, _D)),
                  _full((1, _D)), _full((_D, 256)), _full((1, 256)),
                  _full((1, 256)), _full((1, 256)), _full((256, 128)),
                  _full((1, 128))],
        out_specs=_full((8, 128)),
        out_shape=jax.ShapeDtypeStruct((8, 128), _F32),
    )(h.reshape(_B, _N, _D), fc['W1'].T, fc['b1'][None], fc['g1'][None],
      fc['be1'][None], fc['W2'].T, fc['b2'][None], fc['g2'][None],
      fc['be2'][None], w3p, b3p)

    return logits[:_B, :40]


# speedup vs baseline: 18.3242x; 1.0952x over previous
import functools

import numpy as np
import jax
import jax.numpy as jnp
from jax.experimental import pallas as pl

_B, _N, _D = 4, 2048, 512
_H, _HD, _K = 8, 64, 16
_EPS = 1e-5
_BN = float(1.0 / np.sqrt(1.0 + 1e-5))
_TP = 512
_TA = 256
_F32 = jnp.float32


def _ln(h, g, b):
    mu = jnp.mean(h, axis=-1, keepdims=True)
    c = h - mu
    var = jnp.mean(c * c, axis=-1, keepdims=True)
    return c / jnp.sqrt(var + _EPS) * g + b


def _dot(a, b):
    return jnp.dot(a, b, preferred_element_type=_F32)


def _dot_t(a, b):
    return jax.lax.dot_general(a, b, (((1,), (1,)), ((), ())),
                               preferred_element_type=_F32)


def _embed_body(x_ref, w1_ref, b1_ref, g1_ref, e1_ref, w2_ref, b2_ref,
                g2_ref, e2_ref, o_ref):
    h = _dot(x_ref[...], w1_ref[...]) + b1_ref[...]
    h = jnp.maximum(h * _BN * g1_ref[...] + e1_ref[...], 0.0)
    h = _dot(h, w2_ref[...]) + b2_ref[...]
    h = jnp.maximum(h * _BN * g2_ref[...] + e2_ref[...], 0.0)
    o_ref[...] = h


def _proj_body(hin_ref, win_ref, bin_ref, gin_ref, ein_ref,
               wq_ref, bq_ref, wk_ref, bk_ref, wv_ref, bv_ref,
               h_ref, q_ref, k_ref, v_ref):
    hw = _dot(hin_ref[...].astype(jnp.bfloat16), win_ref[...]) + bin_ref[...]
    h = _ln(hw, gin_ref[...], ein_ref[...])
    h_ref[...] = h
    hb = h.astype(jnp.bfloat16)
    q_ref[...] = (_dot(hb, wq_ref[...]) + bq_ref[...]).astype(jnp.bfloat16)
    k_ref[...] = (_dot(hb, wk_ref[...]) + bk_ref[...]).astype(jnp.bfloat16)
    v_ref[...] = (_dot(hb, wv_ref[...]) + bv_ref[...]).astype(jnp.bfloat16)


def _attn_body(hf_ref, ht_ref, q_ref, kf_ref, vf_ref, ga_ref, ba_ref, o_ref):
    hq = ht_ref[0]
    hall = hf_ref[0]

    sq = hall * hall
    sqrow = _dot_t(jnp.ones((8, _D), _F32), sq)[0:1]
    dist = sqrow - 2.0 * _dot_t(hq, hall)

    q = q_ref[0]
    kb = kf_ref[0]
    vb = vf_ref[0]
    lgs = []
    for hh in range(_H):
        sl = slice(hh * _HD, (hh + 1) * _HD)
        lgs.append(_dot_t(q[:, sl], kb[:, sl]) * (1.0 / 8.0))

    work = dist
    m = jnp.zeros((_TA, 1), _F32)
    for _ in range(_K):
        m = jnp.min(work, axis=1, keepdims=True)
        work = jnp.where(work == m, jnp.float32(np.inf), work)
    msk = dist <= m

    neginf = jnp.float32(-np.inf)
    outs = []
    for hh in range(_H):
        sl = slice(hh * _HD, (hh + 1) * _HD)
        lg = jnp.where(msk, lgs[hh], neginf)
        mx = jnp.max(lg, axis=1, keepdims=True)
        e = jnp.exp(lg - mx)
        p = (e / jnp.sum(e, axis=1, keepdims=True)).astype(jnp.bfloat16)
        outs.append(_dot(p, vb[:, sl]))
    out = jnp.concatenate(outs, axis=1)
    o_ref[0] = _ln(out + hq, ga_ref[...], ba_ref[...])


def _ffn_body(h_ref, w1_ref, b1_ref, w2_ref, b2_ref, g_ref, e_ref, o_ref):
    h = h_ref[...]
    f = jnp.maximum(_dot(h.astype(jnp.bfloat16), w1_ref[...]) + b1_ref[...],
                    0.0)
    f = _dot(f.astype(jnp.bfloat16), w2_ref[...]) + b2_ref[...]
    o_ref[...] = _ln(f + h, g_ref[...], e_ref[...])


def _head_body(h_ref, w1_ref, b1_ref, g1_ref, e1_ref, w2_ref, b2_ref,
               g2_ref, e2_ref, w3_ref, b3_ref, o_ref):
    gs = [jnp.max(h_ref[b], axis=0, keepdims=True) for b in range(_B)]
    g = jnp.concatenate(gs + gs, axis=0)
    z = _dot(g, w1_ref[...]) + b1_ref[...]
    z = jnp.maximum(z * _BN * g1_ref[...] + e1_ref[...], 0.0)
    z = _dot(z, w2_ref[...]) + b2_ref[...]
    z = jnp.maximum(z * _BN * g2_ref[...] + e2_ref[...], 0.0)
    o_ref[...] = _dot(z, w3_ref[...]) + b3_ref[...]


def _full(shape):
    return pl.BlockSpec(shape, lambda *_: tuple(0 for _ in shape))


def _row(c):
    return pl.BlockSpec((_TP, c), lambda i: (i, 0))


def kernel(x, params):
    emb, blocks, fc = params['emb'], params['blocks'], params['fc']
    rows = _B * _N

    xp = jnp.pad(x.reshape(rows, 3), ((0, 0), (0, 125)))
    w1 = jnp.pad(emb['W1'].T, ((0, 125), (0, 0)))
    h = pl.pallas_call(
        _embed_body,
        grid=(rows // _TP,),
        in_specs=[_row(128), _full((128, 128)), _full((1, 128)),
                  _full((1, 128)), _full((1, 128)), _full((128, 256)),
                  _full((1, 256)), _full((1, 256)), _full((1, 256))],
        out_specs=_row(256),
        out_shape=jax.ShapeDtypeStruct((rows, 256), _F32),
    )(xp, w1, emb['b1'][None], emb['g1'][None], emb['be1'][None],
      emb['W2'].T, emb['b2'][None], emb['g2'][None], emb['be2'][None])

    for blk in blocks:
        c_in = h.shape[-1]
        h_, q, k, v = pl.pallas_call(
            _proj_body,
            grid=(rows // _TP,),
            in_specs=[_row(c_in), _full((c_in, _D)), _full((1, _D)),
                      _full((1, _D)), _full((1, _D)),
                      _full((_D, _D)), _full((1, _D)),
                      _full((_D, _D)), _full((1, _D)),
                      _full((_D, _D)), _full((1, _D))],
            out_specs=[_row(_D)] * 4,
            out_shape=[jax.ShapeDtypeStruct((rows, _D), _F32)] +
                      [jax.ShapeDtypeStruct((rows, _D), jnp.bfloat16)] * 3,
        )(h, blk['Win'].T.astype(jnp.bfloat16), blk['bin'][None],
          blk['g_in'][None], blk['b_in'][None],
          blk['Wq'].T.astype(jnp.bfloat16), blk['bq'][None],
          blk['Wk'].T.astype(jnp.bfloat16), blk['bk'][None],
          blk['Wv'].T.astype(jnp.bfloat16), blk['bv'][None])

        h3 = h_.reshape(_B, _N, _D)
        q3 = q.reshape(_B, _N, _D)
        k3 = k.reshape(_B, _N, _D)
        v3 = v.reshape(_B, _N, _D)

        bfull = pl.BlockSpec((1, _N, _D), lambda b, t: (b, 0, 0))
        btile = pl.BlockSpec((1, _TA, _D), lambda b, t: (b, t, 0))
        bvec = pl.BlockSpec((1, _D), lambda b, t: (0, 0))
        h2 = pl.pallas_call(
            _attn_body,
            grid=(_B, _N // _TA),
            in_specs=[bfull, btile, btile, bfull, bfull, bvec, bvec],
            out_specs=btile,
            out_shape=jax.ShapeDtypeStruct((_B, _N, _D), _F32),
        )(h3, h3, q3, k3, v3, blk['g_attn'][None], blk['b_attn'][None])

        h = pl.pallas_call(
            _ffn_body,
            grid=(rows // _TP,),
            in_specs=[_row(_D), _full((_D, 4 * _D)), _full((1, 4 * _D)),
                      _full((4 * _D, _D)), _full((1, _D)),
                      _full((1, _D)), _full((1, _D))],
            out_specs=_row(_D),
            out_shape=jax.ShapeDtypeStruct((rows, _D), _F32),
        )(h2.reshape(rows, _D), blk['Wf1'].T.astype(jnp.bfloat16),
          blk['bf1'][None], blk['Wf2'].T.astype(jnp.bfloat16),
          blk['bf2'][None], blk['g_ffn'][None], blk['b_ffn'][None])

    w3p = jnp.pad(fc['W3'].T, ((0, 0), (0, 128 - 40)))
    b3p = jnp.pad(fc['b3'][None], ((0, 0), (0, 128 - 40)))
    logits = pl.pallas_call(
        _head_body,
        grid=(1,),
        in_specs=[pl.BlockSpec((_B, _N, _D), lambda i: (0, 0, 0)),
                  _full((_D, _D)), _full((1, _D)), _full((1, _D)),
                  _full((1, _D)), _full((_D, 256)), _full((1, 256)),
                  _full((1, 256)), _full((1, 256)), _full((256, 128)),
                  _full((1, 128))],
        out_specs=_full((8, 128)),
        out_shape=jax.ShapeDtypeStruct((8, 128), _F32),
    )(h.reshape(_B, _N, _D), fc['W1'].T, fc['b1'][None], fc['g1'][None],
      fc['be1'][None], fc['W2'].T, fc['b2'][None], fc['g2'][None],
      fc['be2'][None], w3p, b3p)

    return logits[:_B, :40]
